# baseline (speedup 1.0000x reference)
import functools

import jax
import jax.numpy as jnp
from jax import lax
from jax.experimental import pallas as pl
from jax.experimental.pallas import tpu as pltpu
from jax.experimental.pallas import tpu_sc as plsc

NC = 2
NS = 16
NW = NC * NS
CHUNK = 128


def _sc_mesh():
    return plsc.VectorSubcoreMesh(
        core_axis_name="c", subcore_axis_name="s", num_cores=NC,
        num_subcores=NS)


def _degree_kernel(np_rows, ct, dg_out, dg_in):
    rows_pt = np_rows // NS

    @functools.partial(
        pl.kernel,
        out_type=jax.ShapeDtypeStruct((NC, np_rows, 16), jnp.float32),
        mesh=_sc_mesh(),
        scratch_types=[
            pltpu.VMEM((dg_in, CHUNK), jnp.int32),
            pltpu.VMEM((CHUNK, 16), jnp.float32),
            pltpu.VMEM_SHARED((np_rows, 16), jnp.float32),
            pltpu.SemaphoreType.DMA,
        ],
        compiler_params=pltpu.CompilerParams(use_tc_tiling_on_sc=False),
    )
    def deg_kernel(src_hbm, zeros_hbm, ones_hbm, out_hbm, idx_v, ones_v,
                   acc_sh, ssem):
        c = lax.axis_index("c")
        s = lax.axis_index("s")
        wid = c * NS + s
        pltpu.sync_copy(zeros_hbm.at[pl.ds(s * rows_pt, rows_pt)],
                        acc_sh.at[pl.ds(s * rows_pt, rows_pt)])
        pltpu.sync_copy(ones_hbm, ones_v)
        plsc.subcore_barrier()
        base = wid * ct

        @pl.loop(0, dg_out)
        def _(o):
            pltpu.sync_copy(src_hbm.at[pl.ds(base + o * dg_in, dg_in)],
                            idx_v)

            @pl.loop(0, dg_in)
            def _(j):
                pltpu.async_copy(ones_v, acc_sh.at[idx_v.at[j]], ssem,
                                 add=True)

            @pl.loop(0, dg_in)
            def _(j):
                pltpu.make_async_copy(ones_v, acc_sh.at[idx_v.at[0]],
                                      ssem).wait()

        plsc.subcore_barrier()
        pltpu.sync_copy(acc_sh.at[pl.ds(s * rows_pt, rows_pt)],
                        out_hbm.at[c, pl.ds(s * rows_pt, rows_pt)])

    return deg_kernel


def _aggregate_kernel(np_rows, ct, ct_out, ct_in):
    rows_pt = np_rows // NS
    ngrp = ct_in // 4

    @functools.partial(
        pl.kernel,
        out_type=jax.ShapeDtypeStruct((NC, np_rows, 16), jnp.float32),
        mesh=_sc_mesh(),
        scratch_types=[
            pltpu.VMEM((2, ct_in, CHUNK), jnp.int32),
            pltpu.VMEM((2, ct_in, CHUNK), jnp.int32),
            pltpu.VMEM((4, CHUNK, 16), jnp.float32),
            pltpu.VMEM_SHARED((np_rows, 16), jnp.float32),
            [pltpu.SemaphoreType.DMA] * 4,
            [pltpu.SemaphoreType.DMA] * 4,
            pltpu.SemaphoreType.DMA,
            pltpu.SemaphoreType.DMA,
        ],
        compiler_params=pltpu.CompilerParams(use_tc_tiling_on_sc=False),
    )
    def agg_kernel(table_hbm, src_hbm, dst_hbm, zeros_hbm, out_hbm,
                   sidx_v, didx_v, rows_v, acc_sh, gsems, ssems,
                   isem_s, isem_d):
        c = lax.axis_index("c")
        s = lax.axis_index("s")
        wid = c * NS + s
        pltpu.sync_copy(zeros_hbm.at[pl.ds(s * rows_pt, rows_pt)],
                        acc_sh.at[pl.ds(s * rows_pt, rows_pt)])
        plsc.subcore_barrier()
        base = wid * ct

        def load_idx(o, par):
            pltpu.async_copy(src_hbm.at[pl.ds(base + o * ct_in, ct_in)],
                             sidx_v.at[par], isem_s)
            pltpu.async_copy(dst_hbm.at[pl.ds(base + o * ct_in, ct_in)],
                             didx_v.at[par], isem_d)

        def wait_idx():
            pltpu.make_async_copy(src_hbm.at[pl.ds(base, ct_in)],
                                  sidx_v.at[0], isem_s).wait()
            pltpu.make_async_copy(dst_hbm.at[pl.ds(base, ct_in)],
                                  didx_v.at[0], isem_d).wait()

        def gather(p, j, k):
            pltpu.async_copy(table_hbm.at[sidx_v.at[p, j]], rows_v.at[k],
                             gsems[k])

        def scatter(p, j, k):
            pltpu.async_copy(rows_v.at[k], acc_sh.at[didx_v.at[p, j]],
                             ssems[k], add=True)

        def wait_gather(k):
            pltpu.make_async_copy(table_hbm.at[sidx_v.at[0, 0]],
                                  rows_v.at[k], gsems[k]).wait()

        def wait_scatter(k):
            pltpu.make_async_copy(rows_v.at[k], acc_sh.at[didx_v.at[0, 0]],
                                  ssems[k]).wait()

        @pl.loop(0, ct_out)
        def _(o):
            p = lax.rem(o, 2)

            @pl.when(o == 0)
            def _():
                load_idx(o, p)

            wait_idx()
            for k in range(4):
                @pl.when(o > 0)
                def _():
                    wait_scatter(k)

            gather(p, 0, 0)
            gather(p, 1, 1)

            @pl.when(o + 1 < ct_out)
            def _():
                load_idx(o + 1, 1 - p)

            @pl.loop(0, ngrp)
            def _(g):
                for k in range(4):
                    j = 4 * g + k
                    jn = j + 2
                    kn = (k + 2) % 4

                    @pl.when(jn < ct_in)
                    def _():
                        @pl.when(j >= 2)
                        def _():
                            wait_scatter(kn)
                        gather(p, jn, kn)

                    wait_gather(k)
                    scatter(p, j, k)

        for k in range(4):
            wait_scatter(k)
        plsc.subcore_barrier()
        pltpu.sync_copy(acc_sh.at[pl.ds(s * rows_pt, rows_pt)],
                        out_hbm.at[c, pl.ds(s * rows_pt, rows_pt)])

    return agg_kernel


def _split(a):
    hi = a.astype(jnp.bfloat16)
    lo = (a - hi.astype(jnp.float32)).astype(jnp.bfloat16)
    return hi, lo


def _dot_sel(d, s_bf):
    dh, dl = _split(d)
    return (jnp.dot(dh, s_bf, preferred_element_type=jnp.float32)
            + jnp.dot(dl, s_bf, preferred_element_type=jnp.float32))


def _dot_w(t, wh, wl):
    th, tl = _split(t)
    return (jnp.dot(th, wh, preferred_element_type=jnp.float32)
            + jnp.dot(th, wl, preferred_element_type=jnp.float32)
            + jnp.dot(tl, wh, preferred_element_type=jnp.float32))


def _phase_b(degp, featp, s0, gp, blk):
    def body(d0_ref, d1_ref, f_ref, s0_ref, xn_ref, ns_ref):
        d = d0_ref[0] + d1_ref[0]
        nsp = lax.rsqrt(jnp.maximum(_dot_sel(d, s0_ref[...]), 1.0))
        lane = lax.broadcasted_iota(jnp.int32, (blk, 128), 1)
        xn_ref[...] = f_ref[...] * jnp.where(lane % 16 < 10, nsp, 1.0)
        ns_ref[...] = nsp

    spec3a = pl.BlockSpec((1, blk, 128), lambda i: (0, i, 0))
    spec3b = pl.BlockSpec((1, blk, 128), lambda i: (1, i, 0))
    spec = pl.BlockSpec((blk, 128), lambda i: (i, 0))
    cspec = pl.BlockSpec((128, 128), lambda i: (0, 0))
    return pl.pallas_call(
        body,
        grid=(gp // blk,),
        in_specs=[spec3a, spec3b, spec, cspec],
        out_specs=[spec, spec],
        out_shape=[jax.ShapeDtypeStruct((gp, 128), jnp.float32),
                   jax.ShapeDtypeStruct((gp, 128), jnp.float32)],
    )(degp, degp, featp, s0)


def _phase_d(accp, nsp, s10, w1h, w1l, b1big, gp, blk):
    def body(a0_ref, a1_ref, ns_ref, s10_ref, wh_ref, wl_ref, b_ref,
             h_ref, nd_ref):
        sagg = a0_ref[0] + a1_ref[0]
        ndp = lax.rsqrt(jnp.maximum(_dot_sel(sagg, s10_ref[...]), 1.0))
        h = _dot_w(sagg * ndp, wh_ref[...], wl_ref[...])
        h = jnp.maximum(h + b_ref[0:1, :], 0.0)
        h_ref[...] = h * ns_ref[...]
        nd_ref[...] = ndp

    spec3a = pl.BlockSpec((1, blk, 128), lambda i: (0, i, 0))
    spec3b = pl.BlockSpec((1, blk, 128), lambda i: (1, i, 0))
    spec = pl.BlockSpec((blk, 128), lambda i: (i, 0))
    cspec = pl.BlockSpec((128, 128), lambda i: (0, 0))
    bspec = pl.BlockSpec((8, 128), lambda i: (0, 0))
    return pl.pallas_call(
        body,
        grid=(gp // blk,),
        in_specs=[spec3a, spec3b, spec, cspec, cspec, cspec, bspec],
        out_specs=[spec, spec],
        out_shape=[jax.ShapeDtypeStruct((gp, 128), jnp.float32),
                   jax.ShapeDtypeStruct((gp, 128), jnp.float32)],
    )(accp, accp, nsp, s10, w1h, w1l, b1big)


def _phase_f(accp, ndp, w2h, w2l, b2big, gp, blk):
    def body(a0_ref, a1_ref, nd_ref, wh_ref, wl_ref, b_ref, o_ref):
        sagg = (a0_ref[0] + a1_ref[0]) * nd_ref[...]
        o_ref[...] = _dot_w(sagg, wh_ref[...], wl_ref[...]) + b_ref[0:1, :]

    spec3a = pl.BlockSpec((1, blk, 128), lambda i: (0, i, 0))
    spec3b = pl.BlockSpec((1, blk, 128), lambda i: (1, i, 0))
    spec = pl.BlockSpec((blk, 128), lambda i: (i, 0))
    cspec = pl.BlockSpec((128, 128), lambda i: (0, 0))
    bspec = pl.BlockSpec((8, 128), lambda i: (0, 0))
    return pl.pallas_call(
        body,
        grid=(gp // blk,),
        in_specs=[spec3a, spec3b, spec, cspec, cspec, bspec],
        out_specs=spec,
        out_shape=jax.ShapeDtypeStruct((gp, 128), jnp.float32),
    )(accp, accp, ndp, w2h, w2l, b2big)


@jax.jit
def kernel(features, edge_index, W1, b1, W2, b2):
    n = features.shape[0]
    e = edge_index.shape[1]

    np_rows = ((n + 1 + 16 * 8 - 1) // (16 * 8)) * (16 * 8)
    while (np_rows // NS) % 8 != 0:
        np_rows += 16
    ct = -(-e // (NW * CHUNK))
    ct_in = 40
    ct = -(-ct // ct_in) * ct_in
    ct_out = ct // ct_in
    dg_in = 200
    dg_out = ct // dg_in
    ep = NW * ct * CHUNK
    pad = ep - e

    src = edge_index[0]
    dst = edge_index[1]
    padv = jnp.full((pad,), n, jnp.int32)
    src_p = jnp.concatenate([src, padv]).reshape(ep // CHUNK, CHUNK)
    dst_p = jnp.concatenate([dst, padv]).reshape(ep // CHUNK, CHUNK)

    zeros_hbm = jnp.zeros((np_rows, 16), jnp.float32)
    ones_hbm = jnp.ones((CHUNK, 16), jnp.float32)

    gp = np_rows // 8
    blk = gp // 4

    ii = jnp.arange(128)[:, None]
    jj = jnp.arange(128)[None, :]
    s0 = ((jj // 16) * 16 == ii).astype(jnp.bfloat16)
    s10 = ((jj // 16) * 16 + 10 == ii).astype(jnp.bfloat16)
    w1p = jnp.zeros((16, 16), jnp.float32).at[:10, :].set(W1)
    w1big = jnp.kron(jnp.eye(8, dtype=jnp.float32), w1p)
    w1h = w1big.astype(jnp.bfloat16)
    w1l = (w1big - w1h.astype(jnp.float32)).astype(jnp.bfloat16)
    b1big = jnp.broadcast_to(jnp.tile(b1, 8).reshape(1, 128), (8, 128))
    w2p = jnp.zeros((16, 16), jnp.float32).at[:, :2].set(W2)
    w2big = jnp.kron(jnp.eye(8, dtype=jnp.float32), w2p)
    w2h = w2big.astype(jnp.bfloat16)
    w2l = (w2big - w2h.astype(jnp.float32)).astype(jnp.bfloat16)
    b2big = jnp.broadcast_to(
        jnp.tile(jnp.zeros((16,), jnp.float32).at[:2].set(b2),
                 8).reshape(1, 128), (8, 128))

    featp = jnp.concatenate(
        [features, jnp.ones((n, 1), jnp.float32),
         jnp.zeros((n, 5), jnp.float32)], axis=1)
    featp = jnp.concatenate(
        [featp, jnp.zeros((np_rows - n, 16), jnp.float32)]).reshape(gp, 128)

    degp = _degree_kernel(np_rows, ct, dg_out, dg_in)(
        src_p, zeros_hbm, ones_hbm)
    degpp = degp.reshape(NC, gp, 128)

    xnp_, nsp = _phase_b(degpp, featp, s0, gp, blk)

    agg_fn = _aggregate_kernel(np_rows, ct, ct_out, ct_in)
    acc1 = agg_fn(xnp_.reshape(np_rows, 16), src_p, dst_p, zeros_hbm)

    h1np, ndp = _phase_d(acc1.reshape(NC, gp, 128), nsp, s10, w1h, w1l,
                         b1big, gp, blk)

    acc2 = agg_fn(h1np.reshape(np_rows, 16), src_p, dst_p, zeros_hbm)

    outp = _phase_f(acc2.reshape(NC, gp, 128), ndp, w2h, w2l, b2big, gp, blk)
    return outp.reshape(np_rows, 16)[:n, :2]


# speedup vs baseline: 1.8385x; 1.8385x over previous
import functools

import jax
import jax.numpy as jnp
from jax import lax
from jax.experimental import pallas as pl
from jax.experimental.pallas import tpu as pltpu
from jax.experimental.pallas import tpu_sc as plsc

NC = 2
NS = 16
NW = NC * NS
CHUNK = 128


def _sc_mesh():
    return plsc.VectorSubcoreMesh(
        core_axis_name="c", subcore_axis_name="s", num_cores=NC,
        num_subcores=NS)


def _degree_kernel(np_rows, ct, dg_out, dg_in):
    rows_pt = np_rows // NS

    @functools.partial(
        pl.kernel,
        out_type=jax.ShapeDtypeStruct((NC, np_rows, 16), jnp.float32),
        mesh=_sc_mesh(),
        scratch_types=[
            pltpu.VMEM((dg_in, CHUNK), jnp.int32),
            pltpu.VMEM((CHUNK, 16), jnp.float32),
            pltpu.VMEM_SHARED((np_rows, 16), jnp.float32),
            pltpu.SemaphoreType.DMA,
        ],
        compiler_params=pltpu.CompilerParams(use_tc_tiling_on_sc=False),
    )
    def deg_kernel(src_hbm, zeros_hbm, ones_hbm, out_hbm, idx_v, ones_v,
                   acc_sh, ssem):
        c = lax.axis_index("c")
        s = lax.axis_index("s")
        wid = c * NS + s
        pltpu.sync_copy(zeros_hbm.at[pl.ds(s * rows_pt, rows_pt)],
                        acc_sh.at[pl.ds(s * rows_pt, rows_pt)])
        pltpu.sync_copy(ones_hbm, ones_v)
        plsc.subcore_barrier()
        base = wid * ct

        @pl.loop(0, dg_out)
        def _(o):
            pltpu.sync_copy(src_hbm.at[pl.ds(base + o * dg_in, dg_in)],
                            idx_v)

            @pl.loop(0, dg_in)
            def _(j):
                pltpu.async_copy(ones_v, acc_sh.at[idx_v.at[j]], ssem,
                                 add=True)

            @pl.loop(0, dg_in)
            def _(j):
                pltpu.make_async_copy(ones_v, acc_sh.at[idx_v.at[0]],
                                      ssem).wait()

        plsc.subcore_barrier()
        pltpu.sync_copy(acc_sh.at[pl.ds(s * rows_pt, rows_pt)],
                        out_hbm.at[c, pl.ds(s * rows_pt, rows_pt)])

    return deg_kernel


def _aggregate_kernel(np_rows, ct, ct_out, ct_in):
    rows_pt = np_rows // NS
    ngrp = ct_in // 4

    @functools.partial(
        pl.kernel,
        out_type=jax.ShapeDtypeStruct((NC, np_rows, 16), jnp.float32),
        mesh=_sc_mesh(),
        scratch_types=[
            pltpu.VMEM((ct_in, CHUNK), jnp.int32),
            pltpu.VMEM((ct_in, CHUNK), jnp.int32),
            pltpu.VMEM((4, CHUNK, 16), jnp.float32),
            pltpu.VMEM_SHARED((np_rows, 16), jnp.float32),
            [pltpu.SemaphoreType.DMA] * 4,
            [pltpu.SemaphoreType.DMA] * 4,
        ],
        compiler_params=pltpu.CompilerParams(use_tc_tiling_on_sc=False),
    )
    def agg_kernel(table_hbm, src_hbm, dst_hbm, zeros_hbm, out_hbm,
                   sidx_v, didx_v, rows_v, acc_sh, gsems, ssems):
        c = lax.axis_index("c")
        s = lax.axis_index("s")
        wid = c * NS + s
        pltpu.sync_copy(zeros_hbm.at[pl.ds(s * rows_pt, rows_pt)],
                        acc_sh.at[pl.ds(s * rows_pt, rows_pt)])
        plsc.subcore_barrier()
        base = wid * ct

        def gather(j, k):
            pltpu.async_copy(table_hbm.at[sidx_v.at[j]], rows_v.at[k],
                             gsems[k])

        def scatter(j, k):
            pltpu.async_copy(rows_v.at[k], acc_sh.at[didx_v.at[j]],
                             ssems[k], add=True)

        def wait_gather(k):
            pltpu.make_async_copy(table_hbm.at[sidx_v.at[0]], rows_v.at[k],
                                  gsems[k]).wait()

        def wait_scatter(k):
            pltpu.make_async_copy(rows_v.at[k], acc_sh.at[didx_v.at[0]],
                                  ssems[k]).wait()

        @pl.loop(0, ct_out)
        def _(o):
            pltpu.sync_copy(src_hbm.at[pl.ds(base + o * ct_in, ct_in)],
                            sidx_v)
            pltpu.sync_copy(dst_hbm.at[pl.ds(base + o * ct_in, ct_in)],
                            didx_v)
            for k in (0, 1):
                @pl.when(o > 0)
                def _():
                    wait_scatter(k)
                gather(k, k)

            @pl.loop(0, ngrp)
            def _(g):
                for k in range(4):
                    j = 4 * g + k
                    jn = j + 2
                    kn = (k + 2) % 4

                    @pl.when(jn < ct_in)
                    def _():
                        @pl.when((j >= 2) | (o > 0))
                        def _():
                            wait_scatter(kn)
                        gather(jn, kn)

                    wait_gather(k)
                    scatter(j, k)

        for k in range(4):
            wait_scatter(k)
        plsc.subcore_barrier()
        pltpu.sync_copy(acc_sh.at[pl.ds(s * rows_pt, rows_pt)],
                        out_hbm.at[c, pl.ds(s * rows_pt, rows_pt)])

    return agg_kernel


def _split(a):
    hi = a.astype(jnp.bfloat16)
    lo = (a - hi.astype(jnp.float32)).astype(jnp.bfloat16)
    return hi, lo


def _dot_sel(d, s_bf):
    dh, dl = _split(d)
    return (jnp.dot(dh, s_bf, preferred_element_type=jnp.float32)
            + jnp.dot(dl, s_bf, preferred_element_type=jnp.float32))


def _dot_w(t, wh, wl):
    th, tl = _split(t)
    return (jnp.dot(th, wh, preferred_element_type=jnp.float32)
            + jnp.dot(th, wl, preferred_element_type=jnp.float32)
            + jnp.dot(tl, wh, preferred_element_type=jnp.float32))


def _phase_b(degp, featp, s0, gp, blk):
    def body(d0_ref, d1_ref, f_ref, s0_ref, xn_ref, ns_ref):
        d = d0_ref[0] + d1_ref[0]
        nsp = lax.rsqrt(jnp.maximum(_dot_sel(d, s0_ref[...]), 1.0))
        lane = lax.broadcasted_iota(jnp.int32, (blk, 128), 1)
        xn_ref[...] = f_ref[...] * jnp.where(lane % 16 < 10, nsp, 1.0)
        ns_ref[...] = nsp

    spec3a = pl.BlockSpec((1, blk, 128), lambda i: (0, i, 0))
    spec3b = pl.BlockSpec((1, blk, 128), lambda i: (1, i, 0))
    spec = pl.BlockSpec((blk, 128), lambda i: (i, 0))
    cspec = pl.BlockSpec((128, 128), lambda i: (0, 0))
    return pl.pallas_call(
        body,
        grid=(gp // blk,),
        in_specs=[spec3a, spec3b, spec, cspec],
        out_specs=[spec, spec],
        out_shape=[jax.ShapeDtypeStruct((gp, 128), jnp.float32),
                   jax.ShapeDtypeStruct((gp, 128), jnp.float32)],
    )(degp, degp, featp, s0)


def _phase_d(accp, nsp, s10, w1h, w1l, b1big, gp, blk):
    def body(a0_ref, a1_ref, ns_ref, s10_ref, wh_ref, wl_ref, b_ref,
             h_ref, nd_ref):
        sagg = a0_ref[0] + a1_ref[0]
        ndp = lax.rsqrt(jnp.maximum(_dot_sel(sagg, s10_ref[...]), 1.0))
        h = _dot_w(sagg * ndp, wh_ref[...], wl_ref[...])
        h = jnp.maximum(h + b_ref[0:1, :], 0.0)
        h_ref[...] = h * ns_ref[...]
        nd_ref[...] = ndp

    spec3a = pl.BlockSpec((1, blk, 128), lambda i: (0, i, 0))
    spec3b = pl.BlockSpec((1, blk, 128), lambda i: (1, i, 0))
    spec = pl.BlockSpec((blk, 128), lambda i: (i, 0))
    cspec = pl.BlockSpec((128, 128), lambda i: (0, 0))
    bspec = pl.BlockSpec((8, 128), lambda i: (0, 0))
    return pl.pallas_call(
        body,
        grid=(gp // blk,),
        in_specs=[spec3a, spec3b, spec, cspec, cspec, cspec, bspec],
        out_specs=[spec, spec],
        out_shape=[jax.ShapeDtypeStruct((gp, 128), jnp.float32),
                   jax.ShapeDtypeStruct((gp, 128), jnp.float32)],
    )(accp, accp, nsp, s10, w1h, w1l, b1big)


def _phase_f(accp, ndp, w2h, w2l, b2big, gp, blk):
    def body(a0_ref, a1_ref, nd_ref, wh_ref, wl_ref, b_ref, o_ref):
        sagg = (a0_ref[0] + a1_ref[0]) * nd_ref[...]
        o_ref[...] = _dot_w(sagg, wh_ref[...], wl_ref[...]) + b_ref[0:1, :]

    spec3a = pl.BlockSpec((1, blk, 128), lambda i: (0, i, 0))
    spec3b = pl.BlockSpec((1, blk, 128), lambda i: (1, i, 0))
    spec = pl.BlockSpec((blk, 128), lambda i: (i, 0))
    cspec = pl.BlockSpec((128, 128), lambda i: (0, 0))
    bspec = pl.BlockSpec((8, 128), lambda i: (0, 0))
    return pl.pallas_call(
        body,
        grid=(gp // blk,),
        in_specs=[spec3a, spec3b, spec, cspec, cspec, bspec],
        out_specs=spec,
        out_shape=jax.ShapeDtypeStruct((gp, 128), jnp.float32),
    )(accp, accp, ndp, w2h, w2l, b2big)


@jax.jit
def kernel(features, edge_index, W1, b1, W2, b2):
    n = features.shape[0]
    e = edge_index.shape[1]

    np_rows = ((n + 1 + 16 * 8 - 1) // (16 * 8)) * (16 * 8)
    while (np_rows // NS) % 8 != 0:
        np_rows += 16
    ct = -(-e // (NW * CHUNK))
    ct_in = 56
    ct = -(-ct // ct_in) * ct_in
    ct_out = ct // ct_in
    dg_in = 112
    dg_out = ct // dg_in
    ep = NW * ct * CHUNK
    pad = ep - e

    src = edge_index[0]
    dst = edge_index[1]
    padv = jnp.full((pad,), n, jnp.int32)
    src_p = jnp.concatenate([src, padv]).reshape(ep // CHUNK, CHUNK)
    dst_p = jnp.concatenate([dst, padv]).reshape(ep // CHUNK, CHUNK)

    zeros_hbm = jnp.zeros((np_rows, 16), jnp.float32)
    ones_hbm = jnp.ones((CHUNK, 16), jnp.float32)

    gp = np_rows // 8
    blk = gp // 4

    ii = jnp.arange(128)[:, None]
    jj = jnp.arange(128)[None, :]
    s0 = ((jj // 16) * 16 == ii).astype(jnp.bfloat16)
    s10 = ((jj // 16) * 16 + 10 == ii).astype(jnp.bfloat16)
    w1p = jnp.zeros((16, 16), jnp.float32).at[:10, :].set(W1)
    w1big = jnp.kron(jnp.eye(8, dtype=jnp.float32), w1p)
    w1h = w1big.astype(jnp.bfloat16)
    w1l = (w1big - w1h.astype(jnp.float32)).astype(jnp.bfloat16)
    b1big = jnp.broadcast_to(jnp.tile(b1, 8).reshape(1, 128), (8, 128))
    w2p = jnp.zeros((16, 16), jnp.float32).at[:, :2].set(W2)
    w2big = jnp.kron(jnp.eye(8, dtype=jnp.float32), w2p)
    w2h = w2big.astype(jnp.bfloat16)
    w2l = (w2big - w2h.astype(jnp.float32)).astype(jnp.bfloat16)
    b2big = jnp.broadcast_to(
        jnp.tile(jnp.zeros((16,), jnp.float32).at[:2].set(b2),
                 8).reshape(1, 128), (8, 128))

    featp = jnp.concatenate(
        [features, jnp.ones((n, 1), jnp.float32),
         jnp.zeros((n, 5), jnp.float32)], axis=1)
    featp = jnp.concatenate(
        [featp, jnp.zeros((np_rows - n, 16), jnp.float32)]).reshape(gp, 128)

    degp = _degree_kernel(np_rows, ct, dg_out, dg_in)(
        src_p, zeros_hbm, ones_hbm)
    degpp = degp.reshape(NC, gp, 128)

    xnp_, nsp = _phase_b(degpp, featp, s0, gp, blk)

    agg_fn = _aggregate_kernel(np_rows, ct, ct_out, ct_in)
    acc1 = agg_fn(xnp_.reshape(np_rows, 16), src_p, dst_p, zeros_hbm)

    h1np, ndp = _phase_d(acc1.reshape(NC, gp, 128), nsp, s10, w1h, w1l,
                         b1big, gp, blk)

    acc2 = agg_fn(h1np.reshape(np_rows, 16), src_p, dst_p, zeros_hbm)

    outp = _phase_f(acc2.reshape(NC, gp, 128), ndp, w2h, w2l, b2big, gp, blk)
    return outp.reshape(np_rows, 16)[:n, :2]
